# Initial kernel scaffold; baseline (speedup 1.0000x reference)
import jax
import jax.numpy as jnp
from jax.experimental import pallas as pl


def kernel(x, edge_attr, edge_index, node_types, params):
    raise NotImplementedError("write your pallas kernel here")


# speedup vs baseline: 2.8978x; 2.8978x over previous
import functools

import jax
import jax.numpy as jnp
from jax import lax
from jax.experimental import pallas as pl
from jax.experimental.pallas import tpu as pltpu
from jax.experimental.pallas import tpu_sc as plsc

N = 10000
E = 320000
D_IN = 128
D_NODE = 128
D_EATTR = 4
D_EDGE = 64
N_TYPES = 17

NC = 2
NS = 16
NW = NC * NS

C = 40
NCHUNK = E // C
CPW = NCHUNK // NW
GRP = 5

NPAD = 10240
RPT = NPAD // NS

BN = 2000
BE = 5000

F32 = jnp.float32


def _mesh():
    return plsc.VectorSubcoreMesh(core_axis_name="c", subcore_axis_name="s")


def _sc_gather(UV1, UV2, src, dst):
    @functools.partial(
        pl.kernel,
        out_type=[jax.ShapeDtypeStruct((E, D_NODE), F32),
                  jax.ShapeDtypeStruct((E, D_NODE), F32)],
        mesh=_mesh(),
        scratch_types=(
            [pltpu.VMEM((GRP * C,), jnp.int32), pltpu.VMEM((GRP * C,), jnp.int32)]
            + [pltpu.VMEM((C, D_NODE), F32) for _ in range(2 * GRP)]
            + [pltpu.SemaphoreType.DMA, pltpu.SemaphoreType.DMA,
               pltpu.SemaphoreType.DMA]
        ),
    )
    def k(u_hbm, v_hbm, src_hbm, dst_hbm, g1_hbm, g2_hbm, *rest):
        src_v, dst_v = rest[0], rest[1]
        bufs = rest[2:2 + 2 * GRP]
        isem, gsem, wsem = rest[2 + 2 * GRP:]
        wid = lax.axis_index("s") * NC + lax.axis_index("c")
        ebase = wid * CPW * C

        def group(g, carry):
            e0 = ebase + g * (GRP * C)
            hi0 = pltpu.async_copy(src_hbm.at[pl.ds(e0, GRP * C)], src_v, isem)
            hi1 = pltpu.async_copy(dst_hbm.at[pl.ds(e0, GRP * C)], dst_v, isem)
            hi0.wait()
            hi1.wait()
            hs = []
            for b in range(GRP):
                iu = src_v.at[pl.ds(b * C, C)]
                iv = dst_v.at[pl.ds(b * C, C)]
                hs.append(pltpu.async_copy(u_hbm.at[iu], bufs[2 * b], gsem))
                hs.append(pltpu.async_copy(v_hbm.at[iv], bufs[2 * b + 1], gsem))
            for h in hs:
                h.wait()
            ws = []
            for b in range(GRP):
                row0 = e0 + b * C
                ws.append(pltpu.async_copy(bufs[2 * b], g1_hbm.at[pl.ds(row0, C)], wsem))
                ws.append(pltpu.async_copy(bufs[2 * b + 1], g2_hbm.at[pl.ds(row0, C)], wsem))
            for h in ws:
                h.wait()
            return carry

        lax.fori_loop(0, CPW // GRP, group, 0)

    return k(UV1, UV2, src, dst)


def _sc_scatter(e_feat, dst, zeros_init):
    @functools.partial(
        pl.kernel,
        out_type=jax.ShapeDtypeStruct((NC, NPAD, D_NODE), F32),
        mesh=_mesh(),
        scratch_types=(
            [pltpu.VMEM((C,), jnp.int32) for _ in range(GRP)]
            + [pltpu.VMEM((GRP * C, D_NODE), F32),
               pltpu.VMEM_SHARED((NPAD, D_NODE), F32),
               pltpu.SemaphoreType.DMA, pltpu.SemaphoreType.DMA]
        ),
    )
    def k(e_hbm, dst_hbm, z_hbm, out_hbm, *rest):
        idxbufs = rest[:GRP]
        dbuf, acc_sh = rest[GRP], rest[GRP + 1]
        isem, ssem = rest[GRP + 2], rest[GRP + 3]
        c = lax.axis_index("c")
        s = lax.axis_index("s")
        wid = s * NC + c
        ebase = wid * CPW * C
        pltpu.sync_copy(z_hbm.at[pl.ds(s * RPT, RPT)], acc_sh.at[pl.ds(s * RPT, RPT)])
        plsc.subcore_barrier()

        def group(g, carry):
            e0 = ebase + g * (GRP * C)
            his = [pltpu.async_copy(dst_hbm.at[pl.ds(e0 + b * C, C)], idxbufs[b], isem)
                   for b in range(GRP)]
            pltpu.sync_copy(e_hbm.at[pl.ds(e0, GRP * C)], dbuf)
            for h in his:
                h.wait()
            hs = []
            for b in range(GRP):
                hs.append(pltpu.async_copy(
                    dbuf.at[pl.ds(b * C, C)], acc_sh.at[idxbufs[b]], ssem,
                    add=True))
            for h in hs:
                h.wait()
            return carry

        lax.fori_loop(0, CPW // GRP, group, 0)
        plsc.subcore_barrier()
        pltpu.sync_copy(acc_sh.at[pl.ds(s * RPT, RPT)],
                        out_hbm.at[c, pl.ds(s * RPT, RPT)])

    return k(e_feat, dst, zeros_init)


def _full(shape):
    nd = len(shape)
    return pl.BlockSpec(shape, lambda i, _n=nd: (0,) * _n)


def _tc_embed(x, nt, Wst, bst, W1sd, W1ds):
    def body(x_ref, nt_ref, wst_ref, bst_ref, w1s_ref, w1d_ref,
             nf_ref, u_ref, v_ref):
        xb = x_ref[...]
        ntb = nt_ref[...]
        acc = jnp.zeros((BN, D_NODE), F32)
        for t in range(N_TYPES):
            yt = jnp.dot(xb, wst_ref[t], preferred_element_type=F32) + bst_ref[t][None]
            acc = acc + jnp.where(ntb == t, yt, 0.0)
        nf_ref[...] = acc
        u_ref[...] = jnp.dot(acc, w1s_ref[...], preferred_element_type=F32)
        v_ref[...] = jnp.dot(acc, w1d_ref[...], preferred_element_type=F32)

    return pl.pallas_call(
        body,
        grid=(N // BN,),
        in_specs=[
            pl.BlockSpec((BN, D_IN), lambda i: (i, 0)),
            pl.BlockSpec((BN, 1), lambda i: (i, 0)),
            _full((N_TYPES, D_IN, D_NODE)),
            _full((N_TYPES, D_NODE)),
            _full((D_NODE, D_NODE)),
            _full((D_NODE, D_NODE)),
        ],
        out_specs=[
            pl.BlockSpec((BN, D_NODE), lambda i: (i, 0)),
            pl.BlockSpec((BN, D_NODE), lambda i: (i, 0)),
            pl.BlockSpec((BN, D_NODE), lambda i: (i, 0)),
        ],
        out_shape=[
            jax.ShapeDtypeStruct((N, D_NODE), F32),
            jax.ShapeDtypeStruct((N, D_NODE), F32),
            jax.ShapeDtypeStruct((N, D_NODE), F32),
        ],
    )(x, nt, Wst, bst, W1sd, W1ds)


def _tc_edge1(G1, G2, ea, We, be, W1e_p, b1_p, W2_pp, b2_p):
    def body(g1_ref, g2_ref, ea_ref, we_ref, be_ref, w1e_ref, b1_ref,
             w2_ref, b2_ref, e_ref):
        wfold = jnp.dot(we_ref[...], w1e_ref[...], preferred_element_type=F32)
        bfold = jnp.dot(be_ref[...], w1e_ref[...], preferred_element_type=F32)
        p = jnp.dot(ea_ref[...], wfold, preferred_element_type=F32) + bfold
        h = jnp.maximum(g1_ref[...] + g2_ref[...] + p + b1_ref[...], 0.0)
        e_ref[...] = jnp.dot(h, w2_ref[...], preferred_element_type=F32) + b2_ref[...]

    return pl.pallas_call(
        body,
        grid=(E // BE,),
        in_specs=[
            pl.BlockSpec((BE, D_NODE), lambda i: (i, 0)),
            pl.BlockSpec((BE, D_NODE), lambda i: (i, 0)),
            pl.BlockSpec((BE, D_EATTR), lambda i: (i, 0)),
            _full((D_EATTR, D_EDGE)),
            _full((1, D_EDGE)),
            _full((D_EDGE, D_NODE)),
            _full((1, D_NODE)),
            _full((D_NODE, D_NODE)),
            _full((1, D_NODE)),
        ],
        out_specs=pl.BlockSpec((BE, D_NODE), lambda i: (i, 0)),
        out_shape=jax.ShapeDtypeStruct((E, D_NODE), F32),
    )(G1, G2, ea, We, be, W1e_p, b1_p, W2_pp, b2_p)


def _tc_edge2(G1, G2, ep, W1e_pp, b1_p, W2_pp, b2_p, Wec1_p, bec1, Wec2, bec2):
    def body(g1_ref, g2_ref, ep_ref, w1e_ref, b1_ref, w2_ref, b2_ref,
             wec1_ref, bec1_ref, wec2_ref, bec2_ref, e_ref, pe_ref):
        p = jnp.dot(ep_ref[...], w1e_ref[...], preferred_element_type=F32)
        h = jnp.maximum(g1_ref[...] + g2_ref[...] + p + b1_ref[...], 0.0)
        e_new = jnp.dot(h, w2_ref[...], preferred_element_type=F32) + b2_ref[...]
        e_ref[...] = e_new
        t = jnp.maximum(jnp.dot(e_new, wec1_ref[...], preferred_element_type=F32)
                        + bec1_ref[...], 0.0)
        pe_ref[...] = jnp.dot(t, wec2_ref[...], preferred_element_type=F32) + bec2_ref[...]

    return pl.pallas_call(
        body,
        grid=(E // BE,),
        in_specs=[
            pl.BlockSpec((BE, D_NODE), lambda i: (i, 0)),
            pl.BlockSpec((BE, D_NODE), lambda i: (i, 0)),
            pl.BlockSpec((BE, D_NODE), lambda i: (i, 0)),
            _full((D_NODE, D_NODE)),
            _full((1, D_NODE)),
            _full((D_NODE, D_NODE)),
            _full((1, D_NODE)),
            _full((D_NODE, 32)),
            _full((1, 32)),
            _full((32, 1)),
            _full((1, 1)),
        ],
        out_specs=[
            pl.BlockSpec((BE, D_NODE), lambda i: (i, 0)),
            pl.BlockSpec((BE, 1), lambda i: (i, 0)),
        ],
        out_shape=[
            jax.ShapeDtypeStruct((E, D_NODE), F32),
            jax.ShapeDtypeStruct((E, 1), F32),
        ],
    )(G1, G2, ep, W1e_pp, b1_p, W2_pp, b2_p, Wec1_p, bec1, Wec2, bec2)


def _tc_edge3(G1, G2, ep, W1e_pp, b1_p, W2_pp, b2_p):
    def body(g1_ref, g2_ref, ep_ref, w1e_ref, b1_ref, w2_ref, b2_ref,
             e_ref, e64_ref):
        p = jnp.dot(ep_ref[...], w1e_ref[...], preferred_element_type=F32)
        h = jnp.maximum(g1_ref[...] + g2_ref[...] + p + b1_ref[...], 0.0)
        e_new = jnp.dot(h, w2_ref[...], preferred_element_type=F32) + b2_ref[...]
        e_ref[...] = e_new
        e64_ref[...] = e_new[:, :D_EDGE]

    return pl.pallas_call(
        body,
        grid=(E // BE,),
        in_specs=[
            pl.BlockSpec((BE, D_NODE), lambda i: (i, 0)),
            pl.BlockSpec((BE, D_NODE), lambda i: (i, 0)),
            pl.BlockSpec((BE, D_NODE), lambda i: (i, 0)),
            _full((D_NODE, D_NODE)),
            _full((1, D_NODE)),
            _full((D_NODE, D_NODE)),
            _full((1, D_NODE)),
        ],
        out_specs=[
            pl.BlockSpec((BE, D_NODE), lambda i: (i, 0)),
            pl.BlockSpec((BE, D_EDGE), lambda i: (i, 0)),
        ],
        out_shape=[
            jax.ShapeDtypeStruct((E, D_NODE), F32),
            jax.ShapeDtypeStruct((E, D_EDGE), F32),
        ],
    )(G1, G2, ep, W1e_pp, b1_p, W2_pp, b2_p)


def _tc_node(nf, a0, a1, WnA, WnB_p, bn, W1sd, W1ds):
    def body(nf_ref, a0_ref, a1_ref, wna_ref, wnb_ref, bn_ref, w1s_ref, w1d_ref,
             o_ref, u_ref, v_ref):
        agg = a0_ref[...] + a1_ref[...]
        nn = (jnp.dot(nf_ref[...], wna_ref[...], preferred_element_type=F32)
              + jnp.dot(agg, wnb_ref[...], preferred_element_type=F32)
              + bn_ref[...])
        o_ref[...] = nn
        u_ref[...] = jnp.dot(nn, w1s_ref[...], preferred_element_type=F32)
        v_ref[...] = jnp.dot(nn, w1d_ref[...], preferred_element_type=F32)

    return pl.pallas_call(
        body,
        grid=(N // BN,),
        in_specs=[
            pl.BlockSpec((BN, D_NODE), lambda i: (i, 0)),
            pl.BlockSpec((BN, D_NODE), lambda i: (i, 0)),
            pl.BlockSpec((BN, D_NODE), lambda i: (i, 0)),
            _full((D_NODE, D_NODE)),
            _full((D_NODE, D_NODE)),
            _full((1, D_NODE)),
            _full((D_NODE, D_NODE)),
            _full((D_NODE, D_NODE)),
        ],
        out_specs=[
            pl.BlockSpec((BN, D_NODE), lambda i: (i, 0)),
            pl.BlockSpec((BN, D_NODE), lambda i: (i, 0)),
            pl.BlockSpec((BN, D_NODE), lambda i: (i, 0)),
        ],
        out_shape=[
            jax.ShapeDtypeStruct((N, D_NODE), F32),
            jax.ShapeDtypeStruct((N, D_NODE), F32),
            jax.ShapeDtypeStruct((N, D_NODE), F32),
        ],
    )(nf, a0, a1, WnA, WnB_p, bn, W1sd, W1ds)


def _tc_node3(nf, a0, a1, WnA, WnB_p, bn, Wnc1, bnc1, Wnc2, bnc2, Wc1, bc1, Wc2, bc2):
    def body(nf_ref, a0_ref, a1_ref, wna_ref, wnb_ref, bn_ref,
             wnc1_ref, bnc1_ref, wnc2_ref, bnc2_ref,
             wc1_ref, bc1_ref, wc2_ref, bc2_ref,
             o_ref, pn_ref, pc_ref):
        agg = a0_ref[...] + a1_ref[...]
        nn = (jnp.dot(nf_ref[...], wna_ref[...], preferred_element_type=F32)
              + jnp.dot(agg, wnb_ref[...], preferred_element_type=F32)
              + bn_ref[...])
        o_ref[...] = nn
        t1 = jnp.maximum(jnp.dot(nn, wnc1_ref[...], preferred_element_type=F32)
                         + bnc1_ref[...], 0.0)
        pn_ref[...] = jnp.dot(t1, wnc2_ref[...], preferred_element_type=F32) + bnc2_ref[...]
        t2 = jnp.maximum(jnp.dot(nn, wc1_ref[...], preferred_element_type=F32)
                         + bc1_ref[...], 0.0)
        pc_ref[...] = jnp.dot(t2, wc2_ref[...], preferred_element_type=F32) + bc2_ref[...]

    return pl.pallas_call(
        body,
        grid=(N // BN,),
        in_specs=[
            pl.BlockSpec((BN, D_NODE), lambda i: (i, 0)),
            pl.BlockSpec((BN, D_NODE), lambda i: (i, 0)),
            pl.BlockSpec((BN, D_NODE), lambda i: (i, 0)),
            _full((D_NODE, D_NODE)),
            _full((D_NODE, D_NODE)),
            _full((1, D_NODE)),
            _full((D_NODE, 32)),
            _full((1, 32)),
            _full((32, 1)),
            _full((1, 1)),
            _full((D_NODE, 32)),
            _full((1, 32)),
            _full((32, 8)),
            _full((1, 8)),
        ],
        out_specs=[
            pl.BlockSpec((BN, D_NODE), lambda i: (i, 0)),
            pl.BlockSpec((BN, 1), lambda i: (i, 0)),
            pl.BlockSpec((BN, 8), lambda i: (i, 0)),
        ],
        out_shape=[
            jax.ShapeDtypeStruct((N, D_NODE), F32),
            jax.ShapeDtypeStruct((N, 1), F32),
            jax.ShapeDtypeStruct((N, 8), F32),
        ],
    )(nf, a0, a1, WnA, WnB_p, bn, Wnc1, bnc1, Wnc2, bnc2, Wc1, bc1, Wc2, bc2)


def kernel(x, edge_attr, edge_index, node_types, params):
    src = edge_index[0]
    dst = edge_index[1]
    nt = node_types.reshape(N, 1)

    Wst = jnp.stack([p[0][0] for p in params['node_mlps']])
    bst = jnp.stack([p[0][1] for p in params['node_mlps']])
    We, be = params['edge_emb'][0]
    (W1, b1), (W2, b2) = params['mp_edge']
    Wn, bn = params['mp_node'][0]
    (Wec1, bec1), (Wec2, bec2) = params['edge_cls']
    (Wnc1, bnc1), (Wnc2, bnc2) = params['node_cls']
    (Wc1, bc1), (Wc2, bc2) = params['cls']

    W1s, W1d, W1e = W1[:D_NODE], W1[D_NODE:2 * D_NODE], W1[2 * D_NODE:]
    WnA, WnB = Wn[:D_NODE], Wn[D_NODE:]
    r1 = lambda v: v.reshape(1, -1)

    zc = jnp.zeros((D_EDGE, D_EDGE), F32)
    zr = jnp.zeros((D_EDGE,), F32)
    W1sd = jnp.concatenate([W1s, W1d], axis=1)
    W1ds = jnp.concatenate([W1d, W1s], axis=1)
    W1e_p = jnp.concatenate([W1e, zc], axis=1)
    W1e_pp = jnp.concatenate([W1e_p, jnp.zeros((D_EDGE, D_NODE), F32)], axis=0)
    W2_pp = jnp.concatenate(
        [jnp.concatenate([W2, zc], axis=1),
         jnp.zeros((D_EDGE, D_NODE), F32)], axis=0)
    b1_p = jnp.concatenate([b1, zr])
    b2_p = jnp.concatenate([b2, zr])
    WnB_p = jnp.concatenate([WnB, jnp.zeros((D_EDGE, D_NODE), F32)], axis=0)
    Wec1_p = jnp.concatenate([Wec1, jnp.zeros((D_EDGE, 32), F32)], axis=0)

    zeros_init = jnp.zeros((NPAD, D_NODE), F32)

    nf, U, V = _tc_embed(x, nt, Wst, bst, W1sd, W1ds)

    G1, G2 = _sc_gather(U, V, src, dst)
    e1 = _tc_edge1(G1, G2, edge_attr, We, r1(be), W1e_p, r1(b1_p), W2_pp, r1(b2_p))
    agg = _sc_scatter(e1, dst, zeros_init)
    nf, U, V = _tc_node(nf, agg[0, :N], agg[1, :N], WnA, WnB_p, r1(bn), W1sd, W1ds)

    G1, G2 = _sc_gather(U, V, src, dst)
    e2, pe = _tc_edge2(G1, G2, e1, W1e_pp, r1(b1_p), W2_pp, r1(b2_p),
                       Wec1_p, r1(bec1), Wec2, r1(bec2))
    agg = _sc_scatter(e2, dst, zeros_init)
    nf, U, V = _tc_node(nf, agg[0, :N], agg[1, :N], WnA, WnB_p, r1(bn), W1sd, W1ds)

    G1, G2 = _sc_gather(U, V, src, dst)
    e3, e3_64 = _tc_edge3(G1, G2, e2, W1e_pp, r1(b1_p), W2_pp, r1(b2_p))
    agg = _sc_scatter(e3, dst, zeros_init)
    nf3, pn, pc = _tc_node3(nf, agg[0, :N], agg[1, :N], WnA, WnB_p, r1(bn),
                            Wnc1, r1(bnc1), Wnc2, r1(bnc2),
                            Wc1, r1(bc1), Wc2, r1(bc2))

    return (pe.reshape(E), pn.reshape(N), pc, nf3, e3_64)


# speedup vs baseline: 3.0180x; 1.0415x over previous
import functools

import jax
import jax.numpy as jnp
from jax import lax
from jax.experimental import pallas as pl
from jax.experimental.pallas import tpu as pltpu
from jax.experimental.pallas import tpu_sc as plsc

N = 10000
E = 320000
D_IN = 128
D_NODE = 128
D_EATTR = 4
D_EDGE = 64
N_TYPES = 17

NC = 2
NS = 16
NW = NC * NS

C = 80
CPW = E // C // NW
GRP = 5
CS = 40
CPWS = E // CS // NW
GRPS = 5

NPAD = 10240
RPT = NPAD // NS

BN = 2000
BE = 5000

F32 = jnp.float32


def _mesh():
    return plsc.VectorSubcoreMesh(core_axis_name="c", subcore_axis_name="s")


def _sc_gather(UV1, UV2, src, dst):
    @functools.partial(
        pl.kernel,
        out_type=[jax.ShapeDtypeStruct((E, D_NODE), F32),
                  jax.ShapeDtypeStruct((E, D_NODE), F32)],
        mesh=_mesh(),
        scratch_types=(
            [pltpu.VMEM((GRP * C,), jnp.int32), pltpu.VMEM((GRP * C,), jnp.int32)]
            + [pltpu.VMEM((C, D_NODE), F32) for _ in range(2 * GRP)]
            + [pltpu.SemaphoreType.DMA, pltpu.SemaphoreType.DMA,
               pltpu.SemaphoreType.DMA]
        ),
    )
    def k(u_hbm, v_hbm, src_hbm, dst_hbm, g1_hbm, g2_hbm, *rest):
        src_v, dst_v = rest[0], rest[1]
        bufs = rest[2:2 + 2 * GRP]
        isem, gsem, wsem = rest[2 + 2 * GRP:]
        wid = lax.axis_index("s") * NC + lax.axis_index("c")
        ebase = wid * CPW * C

        def group(g, carry):
            e0 = ebase + g * (GRP * C)
            hi0 = pltpu.async_copy(src_hbm.at[pl.ds(e0, GRP * C)], src_v, isem)
            hi1 = pltpu.async_copy(dst_hbm.at[pl.ds(e0, GRP * C)], dst_v, isem)
            hi0.wait()
            hi1.wait()
            hs = []
            for b in range(GRP):
                iu = src_v.at[pl.ds(b * C, C)]
                iv = dst_v.at[pl.ds(b * C, C)]
                hs.append(pltpu.async_copy(u_hbm.at[iu], bufs[2 * b], gsem))
                hs.append(pltpu.async_copy(v_hbm.at[iv], bufs[2 * b + 1], gsem))
            for h in hs:
                h.wait()
            ws = []
            for b in range(GRP):
                row0 = e0 + b * C
                ws.append(pltpu.async_copy(bufs[2 * b], g1_hbm.at[pl.ds(row0, C)], wsem))
                ws.append(pltpu.async_copy(bufs[2 * b + 1], g2_hbm.at[pl.ds(row0, C)], wsem))
            for h in ws:
                h.wait()
            return carry

        lax.fori_loop(0, CPW // GRP, group, 0)

    return k(UV1, UV2, src, dst)


def _sc_scatter(e_feat, dst, zeros_init):
    @functools.partial(
        pl.kernel,
        out_type=jax.ShapeDtypeStruct((NC, NPAD, D_NODE), F32),
        mesh=_mesh(),
        scratch_types=(
            [pltpu.VMEM((CS,), jnp.int32) for _ in range(GRPS)]
            + [pltpu.VMEM((GRPS * CS, D_NODE), F32),
               pltpu.VMEM_SHARED((NPAD, D_NODE), F32),
               pltpu.SemaphoreType.DMA, pltpu.SemaphoreType.DMA]
        ),
    )
    def k(e_hbm, dst_hbm, z_hbm, out_hbm, *rest):
        idxbufs = rest[:GRPS]
        dbuf, acc_sh = rest[GRPS], rest[GRPS + 1]
        isem, ssem = rest[GRPS + 2], rest[GRPS + 3]
        c = lax.axis_index("c")
        s = lax.axis_index("s")
        wid = s * NC + c
        ebase = wid * CPWS * CS
        pltpu.sync_copy(z_hbm.at[pl.ds(s * RPT, RPT)], acc_sh.at[pl.ds(s * RPT, RPT)])
        plsc.subcore_barrier()

        def group(g, carry):
            e0 = ebase + g * (GRPS * CS)
            his = [pltpu.async_copy(dst_hbm.at[pl.ds(e0 + b * CS, CS)], idxbufs[b], isem)
                   for b in range(GRPS)]
            pltpu.sync_copy(e_hbm.at[pl.ds(e0, GRPS * CS)], dbuf)
            for h in his:
                h.wait()
            hs = []
            for b in range(GRPS):
                hs.append(pltpu.async_copy(
                    dbuf.at[pl.ds(b * CS, CS)], acc_sh.at[idxbufs[b]], ssem,
                    add=True))
            for h in hs:
                h.wait()
            return carry

        lax.fori_loop(0, CPWS // GRPS, group, 0)
        plsc.subcore_barrier()
        pltpu.sync_copy(acc_sh.at[pl.ds(s * RPT, RPT)],
                        out_hbm.at[c, pl.ds(s * RPT, RPT)])

    return k(e_feat, dst, zeros_init)


def _full(shape):
    nd = len(shape)
    return pl.BlockSpec(shape, lambda i, _n=nd: (0,) * _n)


def _tc_embed(x, nt, Wst, bst, W1sd, W1ds):
    def body(x_ref, nt_ref, wst_ref, bst_ref, w1s_ref, w1d_ref,
             nf_ref, u_ref, v_ref):
        xb = x_ref[...]
        ntb = nt_ref[...]
        acc = jnp.zeros((BN, D_NODE), F32)
        for t in range(N_TYPES):
            yt = jnp.dot(xb, wst_ref[t], preferred_element_type=F32) + bst_ref[t][None]
            acc = acc + jnp.where(ntb == t, yt, 0.0)
        nf_ref[...] = acc
        u_ref[...] = jnp.dot(acc, w1s_ref[...], preferred_element_type=F32)
        v_ref[...] = jnp.dot(acc, w1d_ref[...], preferred_element_type=F32)

    return pl.pallas_call(
        body,
        grid=(N // BN,),
        in_specs=[
            pl.BlockSpec((BN, D_IN), lambda i: (i, 0)),
            pl.BlockSpec((BN, 1), lambda i: (i, 0)),
            _full((N_TYPES, D_IN, D_NODE)),
            _full((N_TYPES, D_NODE)),
            _full((D_NODE, D_NODE)),
            _full((D_NODE, D_NODE)),
        ],
        out_specs=[
            pl.BlockSpec((BN, D_NODE), lambda i: (i, 0)),
            pl.BlockSpec((BN, D_NODE), lambda i: (i, 0)),
            pl.BlockSpec((BN, D_NODE), lambda i: (i, 0)),
        ],
        out_shape=[
            jax.ShapeDtypeStruct((N, D_NODE), F32),
            jax.ShapeDtypeStruct((N, D_NODE), F32),
            jax.ShapeDtypeStruct((N, D_NODE), F32),
        ],
    )(x, nt, Wst, bst, W1sd, W1ds)


def _tc_edge1(G1, G2, ea, We, be, W1e_p, b1_p, W2_pp, b2_p):
    def body(g1_ref, g2_ref, ea_ref, we_ref, be_ref, w1e_ref, b1_ref,
             w2_ref, b2_ref, e_ref):
        wfold = jnp.dot(we_ref[...], w1e_ref[...], preferred_element_type=F32)
        bfold = jnp.dot(be_ref[...], w1e_ref[...], preferred_element_type=F32)
        p = jnp.dot(ea_ref[...], wfold, preferred_element_type=F32) + bfold
        h = jnp.maximum(g1_ref[...] + g2_ref[...] + p + b1_ref[...], 0.0)
        e_ref[...] = jnp.dot(h, w2_ref[...], preferred_element_type=F32) + b2_ref[...]

    return pl.pallas_call(
        body,
        grid=(E // BE,),
        in_specs=[
            pl.BlockSpec((BE, D_NODE), lambda i: (i, 0)),
            pl.BlockSpec((BE, D_NODE), lambda i: (i, 0)),
            pl.BlockSpec((BE, D_EATTR), lambda i: (i, 0)),
            _full((D_EATTR, D_EDGE)),
            _full((1, D_EDGE)),
            _full((D_EDGE, D_NODE)),
            _full((1, D_NODE)),
            _full((D_NODE, D_NODE)),
            _full((1, D_NODE)),
        ],
        out_specs=pl.BlockSpec((BE, D_NODE), lambda i: (i, 0)),
        out_shape=jax.ShapeDtypeStruct((E, D_NODE), F32),
    )(G1, G2, ea, We, be, W1e_p, b1_p, W2_pp, b2_p)


def _tc_edge2(G1, G2, ep, W1e_pp, b1_p, W2_pp, b2_p, Wec1_p, bec1, Wec2, bec2):
    def body(g1_ref, g2_ref, ep_ref, w1e_ref, b1_ref, w2_ref, b2_ref,
             wec1_ref, bec1_ref, wec2_ref, bec2_ref, e_ref, pe_ref):
        p = jnp.dot(ep_ref[...], w1e_ref[...], preferred_element_type=F32)
        h = jnp.maximum(g1_ref[...] + g2_ref[...] + p + b1_ref[...], 0.0)
        e_new = jnp.dot(h, w2_ref[...], preferred_element_type=F32) + b2_ref[...]
        e_ref[...] = e_new
        t = jnp.maximum(jnp.dot(e_new, wec1_ref[...], preferred_element_type=F32)
                        + bec1_ref[...], 0.0)
        pe_ref[...] = jnp.dot(t, wec2_ref[...], preferred_element_type=F32) + bec2_ref[...]

    return pl.pallas_call(
        body,
        grid=(E // BE,),
        in_specs=[
            pl.BlockSpec((BE, D_NODE), lambda i: (i, 0)),
            pl.BlockSpec((BE, D_NODE), lambda i: (i, 0)),
            pl.BlockSpec((BE, D_NODE), lambda i: (i, 0)),
            _full((D_NODE, D_NODE)),
            _full((1, D_NODE)),
            _full((D_NODE, D_NODE)),
            _full((1, D_NODE)),
            _full((D_NODE, 32)),
            _full((1, 32)),
            _full((32, 1)),
            _full((1, 1)),
        ],
        out_specs=[
            pl.BlockSpec((BE, D_NODE), lambda i: (i, 0)),
            pl.BlockSpec((BE, 1), lambda i: (i, 0)),
        ],
        out_shape=[
            jax.ShapeDtypeStruct((E, D_NODE), F32),
            jax.ShapeDtypeStruct((E, 1), F32),
        ],
    )(G1, G2, ep, W1e_pp, b1_p, W2_pp, b2_p, Wec1_p, bec1, Wec2, bec2)


def _tc_edge3(G1, G2, ep, W1e_pp, b1_p, W2_pp, b2_p):
    def body(g1_ref, g2_ref, ep_ref, w1e_ref, b1_ref, w2_ref, b2_ref,
             e_ref, e64_ref):
        p = jnp.dot(ep_ref[...], w1e_ref[...], preferred_element_type=F32)
        h = jnp.maximum(g1_ref[...] + g2_ref[...] + p + b1_ref[...], 0.0)
        e_new = jnp.dot(h, w2_ref[...], preferred_element_type=F32) + b2_ref[...]
        e_ref[...] = e_new
        e64_ref[...] = e_new[:, :D_EDGE]

    return pl.pallas_call(
        body,
        grid=(E // BE,),
        in_specs=[
            pl.BlockSpec((BE, D_NODE), lambda i: (i, 0)),
            pl.BlockSpec((BE, D_NODE), lambda i: (i, 0)),
            pl.BlockSpec((BE, D_NODE), lambda i: (i, 0)),
            _full((D_NODE, D_NODE)),
            _full((1, D_NODE)),
            _full((D_NODE, D_NODE)),
            _full((1, D_NODE)),
        ],
        out_specs=[
            pl.BlockSpec((BE, D_NODE), lambda i: (i, 0)),
            pl.BlockSpec((BE, D_EDGE), lambda i: (i, 0)),
        ],
        out_shape=[
            jax.ShapeDtypeStruct((E, D_NODE), F32),
            jax.ShapeDtypeStruct((E, D_EDGE), F32),
        ],
    )(G1, G2, ep, W1e_pp, b1_p, W2_pp, b2_p)


def _tc_node(nf, a0, a1, WnA, WnB_p, bn, W1sd, W1ds):
    def body(nf_ref, a0_ref, a1_ref, wna_ref, wnb_ref, bn_ref, w1s_ref, w1d_ref,
             o_ref, u_ref, v_ref):
        agg = a0_ref[...] + a1_ref[...]
        nn = (jnp.dot(nf_ref[...], wna_ref[...], preferred_element_type=F32)
              + jnp.dot(agg, wnb_ref[...], preferred_element_type=F32)
              + bn_ref[...])
        o_ref[...] = nn
        u_ref[...] = jnp.dot(nn, w1s_ref[...], preferred_element_type=F32)
        v_ref[...] = jnp.dot(nn, w1d_ref[...], preferred_element_type=F32)

    return pl.pallas_call(
        body,
        grid=(N // BN,),
        in_specs=[
            pl.BlockSpec((BN, D_NODE), lambda i: (i, 0)),
            pl.BlockSpec((BN, D_NODE), lambda i: (i, 0)),
            pl.BlockSpec((BN, D_NODE), lambda i: (i, 0)),
            _full((D_NODE, D_NODE)),
            _full((D_NODE, D_NODE)),
            _full((1, D_NODE)),
            _full((D_NODE, D_NODE)),
            _full((D_NODE, D_NODE)),
        ],
        out_specs=[
            pl.BlockSpec((BN, D_NODE), lambda i: (i, 0)),
            pl.BlockSpec((BN, D_NODE), lambda i: (i, 0)),
            pl.BlockSpec((BN, D_NODE), lambda i: (i, 0)),
        ],
        out_shape=[
            jax.ShapeDtypeStruct((N, D_NODE), F32),
            jax.ShapeDtypeStruct((N, D_NODE), F32),
            jax.ShapeDtypeStruct((N, D_NODE), F32),
        ],
    )(nf, a0, a1, WnA, WnB_p, bn, W1sd, W1ds)


def _tc_node3(nf, a0, a1, WnA, WnB_p, bn, Wnc1, bnc1, Wnc2, bnc2, Wc1, bc1, Wc2, bc2):
    def body(nf_ref, a0_ref, a1_ref, wna_ref, wnb_ref, bn_ref,
             wnc1_ref, bnc1_ref, wnc2_ref, bnc2_ref,
             wc1_ref, bc1_ref, wc2_ref, bc2_ref,
             o_ref, pn_ref, pc_ref):
        agg = a0_ref[...] + a1_ref[...]
        nn = (jnp.dot(nf_ref[...], wna_ref[...], preferred_element_type=F32)
              + jnp.dot(agg, wnb_ref[...], preferred_element_type=F32)
              + bn_ref[...])
        o_ref[...] = nn
        t1 = jnp.maximum(jnp.dot(nn, wnc1_ref[...], preferred_element_type=F32)
                         + bnc1_ref[...], 0.0)
        pn_ref[...] = jnp.dot(t1, wnc2_ref[...], preferred_element_type=F32) + bnc2_ref[...]
        t2 = jnp.maximum(jnp.dot(nn, wc1_ref[...], preferred_element_type=F32)
                         + bc1_ref[...], 0.0)
        pc_ref[...] = jnp.dot(t2, wc2_ref[...], preferred_element_type=F32) + bc2_ref[...]

    return pl.pallas_call(
        body,
        grid=(N // BN,),
        in_specs=[
            pl.BlockSpec((BN, D_NODE), lambda i: (i, 0)),
            pl.BlockSpec((BN, D_NODE), lambda i: (i, 0)),
            pl.BlockSpec((BN, D_NODE), lambda i: (i, 0)),
            _full((D_NODE, D_NODE)),
            _full((D_NODE, D_NODE)),
            _full((1, D_NODE)),
            _full((D_NODE, 32)),
            _full((1, 32)),
            _full((32, 1)),
            _full((1, 1)),
            _full((D_NODE, 32)),
            _full((1, 32)),
            _full((32, 8)),
            _full((1, 8)),
        ],
        out_specs=[
            pl.BlockSpec((BN, D_NODE), lambda i: (i, 0)),
            pl.BlockSpec((BN, 1), lambda i: (i, 0)),
            pl.BlockSpec((BN, 8), lambda i: (i, 0)),
        ],
        out_shape=[
            jax.ShapeDtypeStruct((N, D_NODE), F32),
            jax.ShapeDtypeStruct((N, 1), F32),
            jax.ShapeDtypeStruct((N, 8), F32),
        ],
    )(nf, a0, a1, WnA, WnB_p, bn, Wnc1, bnc1, Wnc2, bnc2, Wc1, bc1, Wc2, bc2)


def kernel(x, edge_attr, edge_index, node_types, params):
    src = edge_index[0]
    dst = edge_index[1]
    nt = node_types.reshape(N, 1)

    Wst = jnp.stack([p[0][0] for p in params['node_mlps']])
    bst = jnp.stack([p[0][1] for p in params['node_mlps']])
    We, be = params['edge_emb'][0]
    (W1, b1), (W2, b2) = params['mp_edge']
    Wn, bn = params['mp_node'][0]
    (Wec1, bec1), (Wec2, bec2) = params['edge_cls']
    (Wnc1, bnc1), (Wnc2, bnc2) = params['node_cls']
    (Wc1, bc1), (Wc2, bc2) = params['cls']

    W1s, W1d, W1e = W1[:D_NODE], W1[D_NODE:2 * D_NODE], W1[2 * D_NODE:]
    WnA, WnB = Wn[:D_NODE], Wn[D_NODE:]
    r1 = lambda v: v.reshape(1, -1)

    zc = jnp.zeros((D_EDGE, D_EDGE), F32)
    zr = jnp.zeros((D_EDGE,), F32)
    W1sd = jnp.concatenate([W1s, W1d], axis=1)
    W1ds = jnp.concatenate([W1d, W1s], axis=1)
    W1e_p = jnp.concatenate([W1e, zc], axis=1)
    W1e_pp = jnp.concatenate([W1e_p, jnp.zeros((D_EDGE, D_NODE), F32)], axis=0)
    W2_pp = jnp.concatenate(
        [jnp.concatenate([W2, zc], axis=1),
         jnp.zeros((D_EDGE, D_NODE), F32)], axis=0)
    b1_p = jnp.concatenate([b1, zr])
    b2_p = jnp.concatenate([b2, zr])
    WnB_p = jnp.concatenate([WnB, jnp.zeros((D_EDGE, D_NODE), F32)], axis=0)
    Wec1_p = jnp.concatenate([Wec1, jnp.zeros((D_EDGE, 32), F32)], axis=0)

    zeros_init = jnp.zeros((NPAD, D_NODE), F32)

    nf, U, V = _tc_embed(x, nt, Wst, bst, W1sd, W1ds)

    G1, G2 = _sc_gather(U, V, src, dst)
    e1 = _tc_edge1(G1, G2, edge_attr, We, r1(be), W1e_p, r1(b1_p), W2_pp, r1(b2_p))
    agg = _sc_scatter(e1, dst, zeros_init)
    nf, U, V = _tc_node(nf, agg[0, :N], agg[1, :N], WnA, WnB_p, r1(bn), W1sd, W1ds)

    G1, G2 = _sc_gather(U, V, src, dst)
    e2, pe = _tc_edge2(G1, G2, e1, W1e_pp, r1(b1_p), W2_pp, r1(b2_p),
                       Wec1_p, r1(bec1), Wec2, r1(bec2))
    agg = _sc_scatter(e2, dst, zeros_init)
    nf, U, V = _tc_node(nf, agg[0, :N], agg[1, :N], WnA, WnB_p, r1(bn), W1sd, W1ds)

    G1, G2 = _sc_gather(U, V, src, dst)
    e3, e3_64 = _tc_edge3(G1, G2, e2, W1e_pp, r1(b1_p), W2_pp, r1(b2_p))
    agg = _sc_scatter(e3, dst, zeros_init)
    nf3, pn, pc = _tc_node3(nf, agg[0, :N], agg[1, :N], WnA, WnB_p, r1(bn),
                            Wnc1, r1(bnc1), Wnc2, r1(bnc2),
                            Wc1, r1(bc1), Wc2, r1(bc2))

    return (pe.reshape(E), pn.reshape(N), pc, nf3, e3_64)
